# baseline (speedup 1.0000x reference)
import functools

import jax
import jax.numpy as jnp
from jax import lax
from jax.experimental import pallas as pl
from jax.experimental.pallas import tpu as pltpu
from jax.experimental.pallas import tpu_sc as plsc

_SITE = 1024
_TLEN = 24
_EMB = 64
_BATCH = 16
_N = _BATCH * _TLEN * _SITE

_NC = 2
_NS = 16
_NW = _NC * _NS
_BPW = _N // _NW
_IDXW = 128
_ROWS_PER_W = _BPW // _IDXW
_CHUNK_ROWS = 4
_CHUNK = _CHUNK_ROWS * _IDXW
_NCHUNK = _ROWS_PER_W // _CHUNK_ROWS


def _gather_chunk(table_hbm, idx_v, rows, sem, c):
  handles = []
  for k in range(_CHUNK_ROWS):
    handles.append(
        pltpu.async_copy(
            table_hbm.at[idx_v.at[c * _CHUNK_ROWS + k]],
            rows.at[pl.ds(k * _IDXW, _IDXW)],
            sem,
        )
    )
  return handles


@functools.partial(
    pl.kernel,
    out_type=(
        jax.ShapeDtypeStruct((_N, _EMB), jnp.float32),
        jax.ShapeDtypeStruct((_N, _EMB), jnp.float32),
    ),
    mesh=plsc.VectorSubcoreMesh(core_axis_name="c", subcore_axis_name="s"),
    scratch_types=[
        pltpu.VMEM((_ROWS_PER_W, _IDXW), jnp.int32),
        pltpu.VMEM((_CHUNK, _EMB), jnp.float32),
        pltpu.VMEM((_CHUNK, _EMB), jnp.float32),
        pltpu.SemaphoreType.DMA,
        pltpu.SemaphoreType.DMA,
        pltpu.SemaphoreType.DMA,
        pltpu.SemaphoreType.DMA,
    ],
    compiler_params=pltpu.CompilerParams(use_tc_tiling_on_sc=False),
)
def _emb_lookup(dow_hbm, m_hbm, week_hbm, minute_hbm, w_out, m_out,
                idx_v, rows_a, rows_b, gs_a, gs_b, os_a, os_b):
  wid = lax.axis_index("s") * _NC + lax.axis_index("c")
  out_base = wid * _BPW

  def run_table(idx_hbm, table_hbm, out_hbm):
    pltpu.sync_copy(idx_hbm.at[pl.ds(wid * _ROWS_PER_W, _ROWS_PER_W)], idx_v)

    @pl.loop(0, _NCHUNK, step=2)
    def _chunks(c):
      ha = _gather_chunk(table_hbm, idx_v, rows_a, gs_a, c)
      hb = _gather_chunk(table_hbm, idx_v, rows_b, gs_b, c + 1)
      for h in ha:
        h.wait()
      oa = pltpu.async_copy(
          rows_a, out_hbm.at[pl.ds(out_base + c * _CHUNK, _CHUNK)], os_a)
      for h in hb:
        h.wait()
      ob = pltpu.async_copy(
          rows_b, out_hbm.at[pl.ds(out_base + (c + 1) * _CHUNK, _CHUNK)], os_b)
      oa.wait()
      ob.wait()

  run_table(dow_hbm, week_hbm, w_out)
  run_table(m_hbm, minute_hbm, m_out)


def kernel(Dow, M, position_table, week_table, minute_table):
  dow2 = Dow.reshape(_N // _IDXW, _IDXW)
  m2 = M.reshape(_N // _IDXW, _IDXW)
  w_flat, m_flat = _emb_lookup(dow2, m2, week_table, minute_table)
  w_emd = w_flat.reshape(_BATCH, _TLEN, _SITE, _EMB)
  m_emd = m_flat.reshape(_BATCH, _TLEN, _SITE, _EMB)
  p_emd = position_table.reshape(1, 1, _SITE, _EMB)
  return (w_emd, m_emd, p_emd)
